# baseline (speedup 1.0000x reference)
import functools

import jax
import jax.numpy as jnp
import numpy as np
from jax import lax
from jax.experimental import pallas as pl
from jax.experimental.pallas import tpu as pltpu
from jax.experimental.pallas import tpu_sc as plsc

N_NODES = 10240
N_EDGES = 22528
NUM_FEATURES = 40
EDGE_DIM = 10
HIDDEN = 512
HEADS = 4
N_GRAPHS = 256
HC = HEADS * HIDDEN

R_NODES = 512
NR = N_NODES // R_NODES
NRC = NR // 2


def _k1_body(x_ref, w_ref, asd_ref, h2_ref, al_ref):
    j = pl.program_id(1)
    h2 = jnp.dot(x_ref[...], w_ref[...], preferred_element_type=jnp.float32)
    h2_ref[...] = h2
    al = jnp.dot(h2, asd_ref[...], preferred_element_type=jnp.float32)

    @pl.when(j == 0)
    def _():
        al_ref[...] = jnp.zeros_like(al_ref)

    al_ref[...] += al


def _linear_logits(x, wf, asd, bn=512, bc=512):
    n, din = x.shape
    grid = (n // bn, HC // bc)
    return pl.pallas_call(
        _k1_body,
        grid=grid,
        in_specs=[
            pl.BlockSpec((bn, din), lambda i, j: (i, 0)),
            pl.BlockSpec((din, bc), lambda i, j: (0, j)),
            pl.BlockSpec((bc, 2 * HEADS), lambda i, j: (j, 0)),
        ],
        out_specs=[
            pl.BlockSpec((bn, bc), lambda i, j: (i, j)),
            pl.BlockSpec((bn, 2 * HEADS), lambda i, j: (i, 0)),
        ],
        out_shape=[
            jax.ShapeDtypeStruct((n, HC), jnp.float32),
            jax.ShapeDtypeStruct((n, 2 * HEADS), jnp.float32),
        ],
    )(x, wf, asd)


NBP = N_EDGES // 16
BPT = NBP // 16
CW = 128
NQ = HC // CW
NQC = NQ // 2
DR = N_NODES // 16


def _sc_agg_body(h2r_hbm, srcb_hbm, dstb_hbm, arep_hbm, out_hbm,
                 rows_v, src_v, dst_v, arr_v, zbuf, acc_sh, sem1, sem2):
    c = lax.axis_index("c")
    s = lax.axis_index("s")

    for i in range(64):
        def _zb(j, carry):
            zbuf[i, pl.ds(j * 16, 16)] = jnp.zeros((16,), jnp.float32)
            return carry
        lax.fori_loop(0, CW // 16, _zb, 0)

    def _zero_acc():
        for k in range(DR // 64):
            pltpu.sync_copy(zbuf, acc_sh.at[pl.ds(s * DR + k * 64, 64)])

    _zero_acc()
    plsc.subcore_barrier()

    for q in range(NQC):
        qg = c * NQC + q
        hq = qg // (NQ // HEADS)

        def _batch(i, carry):
            bidx = s + i * 16
            cp1 = pltpu.async_copy(srcb_hbm.at[bidx], src_v, sem1)
            cp2 = pltpu.async_copy(dstb_hbm.at[bidx], dst_v, sem1)
            cp3 = pltpu.async_copy(arep_hbm.at[hq * NBP + bidx], arr_v, sem1)
            cp1.wait()
            cp2.wait()
            cp3.wait()
            ridx = src_v[...] + qg * N_NODES
            pltpu.async_copy(h2r_hbm.at[ridx], rows_v, sem2).wait()
            for e in range(16):
                av = arr_v[e, :]
                for u in range(CW // 16):
                    rows_v[e, pl.ds(u * 16, 16)] = (
                        rows_v[e, pl.ds(u * 16, 16)] * av)
            pltpu.sync_copy(rows_v, acc_sh.at[dst_v[...]], add=True)
            return carry

        lax.fori_loop(0, BPT, _batch, 0)
        plsc.subcore_barrier()
        pltpu.sync_copy(acc_sh.at[pl.ds(s * DR, DR)],
                        out_hbm.at[pl.ds(qg * N_NODES + s * DR, DR)])
        _zero_acc()
        plsc.subcore_barrier()


def _sc_aggregate(h2r, srcb, dstb, arep):
    mesh = plsc.VectorSubcoreMesh(core_axis_name="c", subcore_axis_name="s")
    return pl.kernel(
        _sc_agg_body,
        out_type=jax.ShapeDtypeStruct((NQ * N_NODES, CW), jnp.float32),
        mesh=mesh,
        scratch_types=[
            pltpu.VMEM((16, CW), jnp.float32),
            pltpu.VMEM((16,), jnp.int32),
            pltpu.VMEM((16,), jnp.int32),
            pltpu.VMEM((16, 16), jnp.float32),
            pltpu.VMEM((64, CW), jnp.float32),
            pltpu.VMEM_SHARED((N_NODES, CW), jnp.float32),
            pltpu.SemaphoreType.DMA,
            pltpu.SemaphoreType.DMA,
        ],
    )(h2r, srcb, dstb, arep)


def _gat_layer(x, src, dst, srcb, dstb, edge_attr,
               W, We, a_s, a_d, a_e, b):
    H, C = a_s.shape
    wf = W.reshape(W.shape[0], H * C)
    eye = jnp.eye(H, dtype=jnp.float32)
    asd_s = (a_s[:, None, :, None] * eye[:, :, None, None]).transpose(
        0, 2, 1, 3).reshape(H * C, H)
    asd_d = (a_d[:, None, :, None] * eye[:, :, None, None]).transpose(
        0, 2, 1, 3).reshape(H * C, H)
    asd = jnp.concatenate([asd_s, asd_d], axis=1)

    h2, al = _linear_logits(x, wf, asd)
    al_s = al[:, :H]
    al_d = al[:, H:]

    ce = jnp.einsum('dhc,hc->dh', We, a_e)
    al_e = edge_attr @ ce

    e = jax.nn.leaky_relu(al_s[src] + al_d[dst] + al_e, 0.2)
    m = jax.ops.segment_max(e, dst, num_segments=N_NODES)
    m = jnp.where(jnp.isfinite(m), m, 0.0)
    ex = jnp.exp(e - m[dst])
    denom = jax.ops.segment_sum(ex, dst, num_segments=N_NODES)
    alpha = ex / jnp.maximum(denom[dst], 1e-16)

    arep = jnp.broadcast_to(
        alpha.T.reshape(H * NBP, 16, 1), (H * NBP, 16, 16))
    h2r = h2.reshape(N_NODES, NQ, CW).transpose(1, 0, 2).reshape(
        NQ * N_NODES, CW)
    msg1r = _sc_aggregate(h2r, srcb, dstb, arep)
    msg1 = msg1r.reshape(NQ, N_NODES, CW).transpose(1, 0, 2).reshape(
        N_NODES, H, C)

    bmat = jax.ops.segment_sum(
        alpha[:, :, None] * edge_attr[:, None, :], dst,
        num_segments=N_NODES)
    term2 = jnp.einsum('nhd,dhc->nhc', bmat, We)

    out = msg1 + term2
    return jnp.mean(out, axis=1) + b


def kernel(x, edge_index, batch_index, edge_attr, params):
    src = edge_index[0]
    dst = edge_index[1]
    srcb = src.reshape(NBP, 16)
    dstb = dst.reshape(NBP, 16)

    h = x
    for l in range(3):
        h = jax.nn.relu(_gat_layer(
            h, src, dst, srcb, dstb, edge_attr,
            params['W%d' % l], params['We%d' % l], params['as%d' % l],
            params['ad%d' % l], params['ae%d' % l], params['b%d' % l]))
    mu = jnp.mean(h, axis=0)
    var = jnp.var(h, axis=0)
    h = (h - mu) / jnp.sqrt(var + 1e-5) * params['bn_g'] + params['bn_b']
    gmax = jax.ops.segment_max(h, batch_index, num_segments=N_GRAPHS)
    gmax = jnp.where(jnp.isfinite(gmax), gmax, 0.0)
    counts = jax.ops.segment_sum(
        jnp.ones((h.shape[0], 1), jnp.float32), batch_index,
        num_segments=N_GRAPHS)
    gmean = jax.ops.segment_sum(
        h, batch_index, num_segments=N_GRAPHS) / jnp.maximum(counts, 1.0)
    g = jnp.concatenate([gmax, gmean], axis=1)
    g = jax.nn.relu(g @ params['fc1_W'] + params['fc1_b'])
    g = jax.nn.relu(g @ params['fc2_W'] + params['fc2_b'])
    return g @ params['fc3_W'] + params['fc3_b']


# speedup vs baseline: 3.6908x; 3.6908x over previous
import functools

import jax
import jax.numpy as jnp
import numpy as np
from jax import lax
from jax.experimental import pallas as pl
from jax.experimental.pallas import tpu as pltpu
from jax.experimental.pallas import tpu_sc as plsc

N_NODES = 10240
N_EDGES = 22528
NUM_FEATURES = 40
EDGE_DIM = 10
HIDDEN = 512
HEADS = 4
N_GRAPHS = 256
HC = HEADS * HIDDEN

NBP = N_EDGES // 16
NBP4 = N_EDGES // 64
CW = 128
NQ = HC // CW
NQC = NQ // 2
DR = N_NODES // 16


def _k1_body(x_ref, w_ref, asd_ref, h2_ref, al_ref, cmax_ref):
    i = pl.program_id(0)
    j = pl.program_id(1)
    h2 = jnp.dot(x_ref[...], w_ref[...], preferred_element_type=jnp.float32)
    h2_ref[...] = h2
    al = jnp.dot(h2, asd_ref[...], preferred_element_type=jnp.float32)

    @pl.when(j == 0)
    def _():
        al_ref[...] = jnp.zeros_like(al_ref)

    al_ref[...] += al

    @pl.when(j == pl.num_programs(1) - 1)
    def _():
        blkmax = jnp.max(al_ref[...], axis=0, keepdims=True)

        @pl.when(i == 0)
        def _():
            cmax_ref[...] = blkmax

        @pl.when(i > 0)
        def _():
            cmax_ref[...] = jnp.maximum(cmax_ref[...], blkmax)


def _linear_logits(x, wf, asd2, bn=512, bc=512):
    n, din = x.shape
    grid = (n // bn, HC // bc)
    return pl.pallas_call(
        _k1_body,
        grid=grid,
        in_specs=[
            pl.BlockSpec((bn, din), lambda i, j: (i, 0)),
            pl.BlockSpec((din, bc), lambda i, j: (0, j)),
            pl.BlockSpec((bc, 256), lambda i, j: (j, 0)),
        ],
        out_specs=[
            pl.BlockSpec((bn, bc), lambda i, j: (i, j)),
            pl.BlockSpec((bn, 256), lambda i, j: (i, 0)),
            pl.BlockSpec((1, 256), lambda i, j: (0, 0)),
        ],
        out_shape=[
            jax.ShapeDtypeStruct((n, HC), jnp.float32),
            jax.ShapeDtypeStruct((n, 256), jnp.float32),
            jax.ShapeDtypeStruct((1, 256), jnp.float32),
        ],
    )(x, wf, asd2)


_GPT = NBP4 // 16
_GPT2 = NBP4 // 32


def _k2_body(als_hbm, ald_hbm, aleb_hbm, srcb_hbm, dstb_hbm, cvec_hbm,
             alpha_hbm, sidx_v, didx_v, gsrc_v, gdst_v, ale_v, exv_v, av_v,
             cvec_v, denom_sh, sem1, sem2):
    c = lax.axis_index("c")
    s = lax.axis_index("s")
    pltpu.sync_copy(cvec_hbm, cvec_v)
    iota16 = lax.broadcasted_iota(jnp.int32, (16,), 0)
    lanemask = jnp.where(iota16 < HEADS, 1.0, 0.0)
    cv = cvec_v[...]

    for i in range(64):
        def _zb(j, carry):
            exv_v[i, pl.ds(j * 16, 16)] = jnp.zeros((16,), jnp.float32)
            return carry
        lax.fori_loop(0, 8, _zb, 0)
    for k in range(DR // 64):
        pltpu.sync_copy(exv_v, denom_sh.at[pl.ds(s * DR + k * 64, 64)])
    plsc.subcore_barrier()

    def _ex_rows(g):
        cp1 = pltpu.async_copy(srcb_hbm.at[g], sidx_v, sem1)
        cp2 = pltpu.async_copy(dstb_hbm.at[g], didx_v, sem1)
        cp3 = pltpu.async_copy(aleb_hbm.at[g], ale_v, sem1)
        cp1.wait()
        cp2.wait()
        cp3.wait()
        cp4 = pltpu.async_copy(als_hbm.at[sidx_v], gsrc_v, sem2)
        cp5 = pltpu.async_copy(ald_hbm.at[didx_v], gdst_v, sem2)
        cp4.wait()
        cp5.wait()
        for e in range(64):
            srow = (gsrc_v[e, pl.ds(0, 16)] + gdst_v[e, pl.ds(0, 16)]
                    + ale_v[e, :])
            srow = jnp.where(srow > 0, srow, srow * 0.2)
            exv_v[e, pl.ds(0, 16)] = jnp.exp(srow - cv) * lanemask

    def _p1(i, carry):
        g = s + i * 16
        _ex_rows(g)
        pltpu.sync_copy(exv_v, denom_sh.at[didx_v], add=True)
        return carry

    lax.fori_loop(0, _GPT, _p1, 0)
    plsc.subcore_barrier()

    def _p2(i, carry):
        g = c * (NBP4 // 2) + s + i * 16
        _ex_rows(g)
        pltpu.sync_copy(denom_sh.at[didx_v], gsrc_v)
        for e in range(64):
            d = jnp.maximum(gsrc_v[e, pl.ds(0, 16)], 1e-16)
            av_v[e, :] = exv_v[e, pl.ds(0, 16)] / d
        pltpu.sync_copy(av_v, alpha_hbm.at[g])
        return carry

    lax.fori_loop(0, _GPT2, _p2, 0)


def _sc_softmax(als128, ald128, aleb, srcb4, dstb4, cvec):
    mesh = plsc.VectorSubcoreMesh(core_axis_name="c", subcore_axis_name="s")
    return pl.kernel(
        _k2_body,
        out_type=jax.ShapeDtypeStruct((NBP4, 64, 16), jnp.float32),
        mesh=mesh,
        scratch_types=[
            pltpu.VMEM((64,), jnp.int32),
            pltpu.VMEM((64,), jnp.int32),
            pltpu.VMEM((64, 128), jnp.float32),
            pltpu.VMEM((64, 128), jnp.float32),
            pltpu.VMEM((64, 16), jnp.float32),
            pltpu.VMEM((64, 128), jnp.float32),
            pltpu.VMEM((64, 16), jnp.float32),
            pltpu.VMEM((16,), jnp.float32),
            pltpu.VMEM_SHARED((N_NODES, 128), jnp.float32),
            pltpu.SemaphoreType.DMA,
            pltpu.SemaphoreType.DMA,
        ],
    )(als128, ald128, aleb, srcb4, dstb4, cvec)


_BPT4 = NBP4 // 16


def _k3_body(h2r_hbm, srcb_hbm, dstb_hbm, arep_hbm, eab_hbm,
             out_hbm, bmatw_hbm,
             rows_v, sidx_v, didx_v, arr_v, arr2_v, ea_v, bout_v,
             acc_sh, sem1, sem2):
    c = lax.axis_index("c")
    s = lax.axis_index("s")

    for i in range(64):
        def _zb(j, carry):
            bout_v[i, pl.ds(j * 16, 16)] = jnp.zeros((16,), jnp.float32)
            return carry
        lax.fori_loop(0, CW // 16, _zb, 0)

    def _zero_acc():
        for k in range(DR // 64):
            pltpu.sync_copy(bout_v, acc_sh.at[pl.ds(s * DR + k * 64, 64)])

    _zero_acc()
    plsc.subcore_barrier()

    for q in range(NQC):
        qg = c * NQC + q
        hq = qg // (NQ // HEADS)

        def _batch(i, carry):
            g = s + i * 16
            cp1 = pltpu.async_copy(srcb_hbm.at[g], sidx_v, sem1)
            cp2 = pltpu.async_copy(dstb_hbm.at[g], didx_v, sem1)
            cp3 = pltpu.async_copy(arep_hbm.at[hq * NBP4 + g], arr_v, sem1)
            cp1.wait()
            cp2.wait()
            cp3.wait()
            for j in range(4):
                sl = pl.ds(j * 16, 16)
                sidx_v[sl] = sidx_v[sl] + qg * N_NODES
            pltpu.async_copy(h2r_hbm.at[sidx_v], rows_v, sem2).wait()
            for e in range(64):
                av = arr_v[e, :]
                for u in range(CW // 16):
                    rows_v[e, pl.ds(u * 16, 16)] = (
                        rows_v[e, pl.ds(u * 16, 16)] * av)
            pltpu.sync_copy(rows_v, acc_sh.at[didx_v], add=True)
            return carry

        lax.fori_loop(0, _BPT4, _batch, 0)
        plsc.subcore_barrier()
        pltpu.sync_copy(acc_sh.at[pl.ds(s * DR, DR)],
                        out_hbm.at[pl.ds(qg * N_NODES + s * DR, DR)])
        _zero_acc()
        plsc.subcore_barrier()

    h0 = 2 * c

    def _bbatch(i, carry):
        g = s + i * 16
        cp1 = pltpu.async_copy(dstb_hbm.at[g], didx_v, sem1)
        cp2 = pltpu.async_copy(arep_hbm.at[h0 * NBP4 + g], arr_v, sem1)
        cp3 = pltpu.async_copy(arep_hbm.at[(h0 + 1) * NBP4 + g], arr2_v, sem1)
        cp4 = pltpu.async_copy(eab_hbm.at[g], ea_v, sem1)
        cp1.wait()
        cp2.wait()
        cp3.wait()
        cp4.wait()
        for e in range(64):
            bout_v[e, pl.ds(0, 16)] = ea_v[e, :] * arr_v[e, :]
            bout_v[e, pl.ds(16, 16)] = ea_v[e, :] * arr2_v[e, :]
        pltpu.sync_copy(bout_v, acc_sh.at[didx_v], add=True)
        return carry

    lax.fori_loop(0, _BPT4, _bbatch, 0)
    plsc.subcore_barrier()
    pltpu.sync_copy(acc_sh.at[pl.ds(s * DR, DR)],
                    bmatw_hbm.at[pl.ds(c * N_NODES + s * DR, DR)])


def _sc_aggregate(h2r, srcb4, dstb4, arep, eab):
    mesh = plsc.VectorSubcoreMesh(core_axis_name="c", subcore_axis_name="s")
    return pl.kernel(
        _k3_body,
        out_type=(
            jax.ShapeDtypeStruct((NQ * N_NODES, CW), jnp.float32),
            jax.ShapeDtypeStruct((2 * N_NODES, CW), jnp.float32),
        ),
        mesh=mesh,
        scratch_types=[
            pltpu.VMEM((64, CW), jnp.float32),
            pltpu.VMEM((64,), jnp.int32),
            pltpu.VMEM((64,), jnp.int32),
            pltpu.VMEM((64, 16), jnp.float32),
            pltpu.VMEM((64, 16), jnp.float32),
            pltpu.VMEM((64, 16), jnp.float32),
            pltpu.VMEM((64, CW), jnp.float32),
            pltpu.VMEM_SHARED((N_NODES, CW), jnp.float32),
            pltpu.SemaphoreType.DMA,
            pltpu.SemaphoreType.DMA,
        ],
    )(h2r, srcb4, dstb4, arep, eab)


def _gat_layer(x, srcb4, dstb4, eab, edge_attr, W, We, a_s, a_d, a_e, b):
    H, C = a_s.shape
    wf = W.reshape(W.shape[0], H * C)
    eye = jnp.eye(H, dtype=jnp.float32)
    asd_s = (a_s[:, None, :, None] * eye[:, :, None, None]).transpose(
        0, 2, 1, 3).reshape(H * C, H)
    asd_d = (a_d[:, None, :, None] * eye[:, :, None, None]).transpose(
        0, 2, 1, 3).reshape(H * C, H)
    z = jnp.zeros((H * C, 128 - H), jnp.float32)
    asd2 = jnp.concatenate([asd_s, z, asd_d, z], axis=1)

    h2, al2, cmax = _linear_logits(x, wf, asd2)
    als128 = al2[:, :128]
    ald128 = al2[:, 128:]

    ce = jnp.einsum('dhc,hc->dh', We, a_e)
    al_e = edge_attr @ ce
    ale_max = jnp.max(al_e, axis=0)

    craw = cmax[0, :H] + cmax[0, 128:128 + H] + ale_max
    cshift = jnp.where(craw > 0, craw, craw * 0.2)
    cvec = jnp.pad(cshift, (0, 12))

    aleb = jnp.pad(al_e, ((0, 0), (0, 12))).reshape(NBP4, 64, 16)

    alpha16 = _sc_softmax(als128, ald128, aleb, srcb4, dstb4, cvec)
    alpha = alpha16.reshape(N_EDGES, 16)[:, :H]

    arep = jnp.broadcast_to(
        alpha.T.reshape(H * NBP4, 64, 1), (H * NBP4, 64, 16))
    h2r = h2.reshape(N_NODES, NQ, CW).transpose(1, 0, 2).reshape(
        NQ * N_NODES, CW)
    msg1r, bmatw = _sc_aggregate(h2r, srcb4, dstb4, arep, eab)
    msg1 = msg1r.reshape(NQ, N_NODES, CW).transpose(1, 0, 2).reshape(
        N_NODES, H, C)

    bmat = bmatw.reshape(2, N_NODES, 8, 16)[:, :, :2, :EDGE_DIM]
    bmat = bmat.transpose(1, 0, 2, 3).reshape(N_NODES, H, EDGE_DIM)
    term2 = jnp.einsum('nhd,dhc->nhc', bmat, We)

    out = msg1 + term2
    return jnp.mean(out, axis=1) + b


def kernel(x, edge_index, batch_index, edge_attr, params):
    src = edge_index[0]
    dst = edge_index[1]
    srcb4 = src.reshape(NBP4, 64)
    dstb4 = dst.reshape(NBP4, 64)
    eab = jnp.pad(edge_attr, ((0, 0), (0, 16 - EDGE_DIM))).reshape(
        NBP4, 64, 16)

    h = x
    for l in range(3):
        h = jax.nn.relu(_gat_layer(
            h, srcb4, dstb4, eab, edge_attr,
            params['W%d' % l], params['We%d' % l], params['as%d' % l],
            params['ad%d' % l], params['ae%d' % l], params['b%d' % l]))
    mu = jnp.mean(h, axis=0)
    var = jnp.var(h, axis=0)
    h = (h - mu) / jnp.sqrt(var + 1e-5) * params['bn_g'] + params['bn_b']
    gmax = jax.ops.segment_max(h, batch_index, num_segments=N_GRAPHS)
    gmax = jnp.where(jnp.isfinite(gmax), gmax, 0.0)
    counts = jax.ops.segment_sum(
        jnp.ones((h.shape[0], 1), jnp.float32), batch_index,
        num_segments=N_GRAPHS)
    gmean = jax.ops.segment_sum(
        h, batch_index, num_segments=N_GRAPHS) / jnp.maximum(counts, 1.0)
    g = jnp.concatenate([gmax, gmean], axis=1)
    g = jax.nn.relu(g @ params['fc1_W'] + params['fc1_b'])
    g = jax.nn.relu(g @ params['fc2_W'] + params['fc2_b'])
    return g @ params['fc3_W'] + params['fc3_b']


# speedup vs baseline: 4.3175x; 1.1698x over previous
import functools

import jax
import jax.numpy as jnp
import numpy as np
from jax import lax
from jax.experimental import pallas as pl
from jax.experimental.pallas import tpu as pltpu
from jax.experimental.pallas import tpu_sc as plsc

N_NODES = 10240
N_EDGES = 22528
NUM_FEATURES = 40
EDGE_DIM = 10
HIDDEN = 512
HEADS = 4
N_GRAPHS = 256
HC = HEADS * HIDDEN

NBP = N_EDGES // 16
NBP4 = N_EDGES // 64
CW = 128
NQ = HC // CW
NQC = NQ // 2
DR = N_NODES // 16


def _k1_body(x_ref, w_ref, asd_ref, h2_ref, al_ref, cmax_ref):
    i = pl.program_id(0)
    j = pl.program_id(1)
    h2 = jnp.dot(x_ref[...], w_ref[...], preferred_element_type=jnp.float32)
    bn = h2.shape[0]
    h2_ref[...] = jnp.transpose(h2.reshape(bn, 4, CW), (1, 0, 2))
    al = jnp.dot(h2, asd_ref[...], preferred_element_type=jnp.float32)

    @pl.when(j == 0)
    def _():
        al_ref[...] = jnp.zeros_like(al_ref)

    al_ref[...] += al

    @pl.when(j == pl.num_programs(1) - 1)
    def _():
        blkmax = jnp.max(al_ref[...], axis=0, keepdims=True)

        @pl.when(i == 0)
        def _():
            cmax_ref[...] = blkmax

        @pl.when(i > 0)
        def _():
            cmax_ref[...] = jnp.maximum(cmax_ref[...], blkmax)


def _linear_logits(x, wf, asd2, bn=512, bc=512):
    n, din = x.shape
    grid = (n // bn, HC // bc)
    return pl.pallas_call(
        _k1_body,
        grid=grid,
        in_specs=[
            pl.BlockSpec((bn, din), lambda i, j: (i, 0)),
            pl.BlockSpec((din, bc), lambda i, j: (0, j)),
            pl.BlockSpec((bc, 256), lambda i, j: (j, 0)),
        ],
        out_specs=[
            pl.BlockSpec((4, bn, CW), lambda i, j: (j, i, 0)),
            pl.BlockSpec((bn, 256), lambda i, j: (i, 0)),
            pl.BlockSpec((1, 256), lambda i, j: (0, 0)),
        ],
        out_shape=[
            jax.ShapeDtypeStruct((NQ, n, CW), jnp.float32),
            jax.ShapeDtypeStruct((n, 256), jnp.float32),
            jax.ShapeDtypeStruct((1, 256), jnp.float32),
        ],
    )(x, wf, asd2)


_GPT = NBP4 // 16
_GPT2 = NBP4 // 32


def _k2_body(als_hbm, ald_hbm, aleb_hbm, srcb_hbm, dstb_hbm, cvec_hbm,
             alpha_hbm, sidx_v, didx_v, gsrc_v, gdst_v, ale_v, exv_v, av_v,
             cvec_v, denom_sh, sem1, sem2):
    c = lax.axis_index("c")
    s = lax.axis_index("s")
    pltpu.sync_copy(cvec_hbm, cvec_v)
    iota16 = lax.broadcasted_iota(jnp.int32, (16,), 0)
    lanemask = jnp.where(iota16 < HEADS, 1.0, 0.0)
    cv = cvec_v[...]

    for i in range(64):
        def _zb(j, carry):
            exv_v[i, pl.ds(j * 16, 16)] = jnp.zeros((16,), jnp.float32)
            return carry
        lax.fori_loop(0, 8, _zb, 0)
    for k in range(DR // 64):
        pltpu.sync_copy(exv_v, denom_sh.at[pl.ds(s * DR + k * 64, 64)])
    plsc.subcore_barrier()

    def _ex_rows(g):
        cp1 = pltpu.async_copy(srcb_hbm.at[g], sidx_v, sem1)
        cp2 = pltpu.async_copy(dstb_hbm.at[g], didx_v, sem1)
        cp3 = pltpu.async_copy(aleb_hbm.at[g], ale_v, sem1)
        cp1.wait()
        cp2.wait()
        cp3.wait()
        cp4 = pltpu.async_copy(als_hbm.at[sidx_v], gsrc_v, sem2)
        cp5 = pltpu.async_copy(ald_hbm.at[didx_v], gdst_v, sem2)
        cp4.wait()
        cp5.wait()
        for e in range(64):
            srow = (gsrc_v[e, pl.ds(0, 16)] + gdst_v[e, pl.ds(0, 16)]
                    + ale_v[e, :])
            srow = jnp.where(srow > 0, srow, srow * 0.2)
            exv_v[e, pl.ds(0, 16)] = jnp.exp(srow - cv) * lanemask

    def _p1(i, carry):
        g = s + i * 16
        _ex_rows(g)
        pltpu.sync_copy(exv_v, denom_sh.at[didx_v], add=True)
        return carry

    lax.fori_loop(0, _GPT, _p1, 0)
    plsc.subcore_barrier()

    def _p2(i, carry):
        g = c * (NBP4 // 2) + s + i * 16
        _ex_rows(g)
        pltpu.sync_copy(denom_sh.at[didx_v], gsrc_v)
        for e in range(64):
            d = jnp.maximum(gsrc_v[e, pl.ds(0, 16)], 1e-16)
            av_v[e, :] = exv_v[e, pl.ds(0, 16)] / d
        pltpu.sync_copy(av_v, alpha_hbm.at[g])
        return carry

    lax.fori_loop(0, _GPT2, _p2, 0)


def _sc_softmax(als128, ald128, aleb, srcb4, dstb4, cvec):
    mesh = plsc.VectorSubcoreMesh(core_axis_name="c", subcore_axis_name="s")
    return pl.kernel(
        _k2_body,
        out_type=jax.ShapeDtypeStruct((NBP4, 64, 16), jnp.float32),
        mesh=mesh,
        scratch_types=[
            pltpu.VMEM((64,), jnp.int32),
            pltpu.VMEM((64,), jnp.int32),
            pltpu.VMEM((64, 128), jnp.float32),
            pltpu.VMEM((64, 128), jnp.float32),
            pltpu.VMEM((64, 16), jnp.float32),
            pltpu.VMEM((64, 128), jnp.float32),
            pltpu.VMEM((64, 16), jnp.float32),
            pltpu.VMEM((16,), jnp.float32),
            pltpu.VMEM_SHARED((N_NODES, 128), jnp.float32),
            pltpu.SemaphoreType.DMA,
            pltpu.SemaphoreType.DMA,
        ],
    )(als128, ald128, aleb, srcb4, dstb4, cvec)


_BPT4 = NBP4 // 16


def _k3_body(h2r_hbm, srcb_hbm, dstb_hbm, arep_hbm, eab_hbm,
             out_hbm, bmatw_hbm,
             rows_v, sidx_v, didx_v, arr_v, arr2_v, ea_v, bout_v,
             acc_sh, sem1, sem2):
    c = lax.axis_index("c")
    s = lax.axis_index("s")

    for i in range(64):
        def _zb(j, carry):
            bout_v[i, pl.ds(j * 16, 16)] = jnp.zeros((16,), jnp.float32)
            return carry
        lax.fori_loop(0, CW // 16, _zb, 0)

    def _zero_acc():
        for k in range(DR // 64):
            pltpu.sync_copy(bout_v, acc_sh.at[pl.ds(s * DR + k * 64, 64)])

    _zero_acc()
    plsc.subcore_barrier()

    for q in range(NQC):
        qg = c * NQC + q
        hq = qg // (NQ // HEADS)

        def _batch(i, carry):
            g = s + i * 16
            cp1 = pltpu.async_copy(srcb_hbm.at[g], sidx_v, sem1)
            cp2 = pltpu.async_copy(dstb_hbm.at[g], didx_v, sem1)
            cp3 = pltpu.async_copy(arep_hbm.at[hq * NBP4 + g], arr_v, sem1)
            cp1.wait()
            cp2.wait()
            cp3.wait()
            for j in range(4):
                sl = pl.ds(j * 16, 16)
                sidx_v[sl] = sidx_v[sl] + qg * N_NODES
            pltpu.async_copy(h2r_hbm.at[sidx_v], rows_v, sem2).wait()
            for e in range(64):
                av = arr_v[e, :]
                for u in range(CW // 16):
                    rows_v[e, pl.ds(u * 16, 16)] = (
                        rows_v[e, pl.ds(u * 16, 16)] * av)
            pltpu.sync_copy(rows_v, acc_sh.at[didx_v], add=True)
            return carry

        lax.fori_loop(0, _BPT4, _batch, 0)
        plsc.subcore_barrier()
        pltpu.sync_copy(acc_sh.at[pl.ds(s * DR, DR)],
                        out_hbm.at[pl.ds(qg * N_NODES + s * DR, DR)])
        _zero_acc()
        plsc.subcore_barrier()

    h0 = 2 * c

    def _bbatch(i, carry):
        g = s + i * 16
        cp1 = pltpu.async_copy(dstb_hbm.at[g], didx_v, sem1)
        cp2 = pltpu.async_copy(arep_hbm.at[h0 * NBP4 + g], arr_v, sem1)
        cp3 = pltpu.async_copy(arep_hbm.at[(h0 + 1) * NBP4 + g], arr2_v, sem1)
        cp4 = pltpu.async_copy(eab_hbm.at[g], ea_v, sem1)
        cp1.wait()
        cp2.wait()
        cp3.wait()
        cp4.wait()
        for e in range(64):
            bout_v[e, pl.ds(0, 16)] = ea_v[e, :] * arr_v[e, :]
            bout_v[e, pl.ds(16, 16)] = ea_v[e, :] * arr2_v[e, :]
        pltpu.sync_copy(bout_v, acc_sh.at[didx_v], add=True)
        return carry

    lax.fori_loop(0, _BPT4, _bbatch, 0)
    plsc.subcore_barrier()
    pltpu.sync_copy(acc_sh.at[pl.ds(s * DR, DR)],
                    bmatw_hbm.at[pl.ds(c * N_NODES + s * DR, DR)])


def _sc_aggregate(h2r, srcb4, dstb4, arep, eab):
    mesh = plsc.VectorSubcoreMesh(core_axis_name="c", subcore_axis_name="s")
    return pl.kernel(
        _k3_body,
        out_type=(
            jax.ShapeDtypeStruct((NQ * N_NODES, CW), jnp.float32),
            jax.ShapeDtypeStruct((2 * N_NODES, CW), jnp.float32),
        ),
        mesh=mesh,
        scratch_types=[
            pltpu.VMEM((64, CW), jnp.float32),
            pltpu.VMEM((64,), jnp.int32),
            pltpu.VMEM((64,), jnp.int32),
            pltpu.VMEM((64, 16), jnp.float32),
            pltpu.VMEM((64, 16), jnp.float32),
            pltpu.VMEM((64, 16), jnp.float32),
            pltpu.VMEM((64, CW), jnp.float32),
            pltpu.VMEM_SHARED((N_NODES, CW), jnp.float32),
            pltpu.SemaphoreType.DMA,
            pltpu.SemaphoreType.DMA,
        ],
    )(h2r, srcb4, dstb4, arep, eab)


def _k4_body(msg_ref, b64_ref, wbm_ref, bias_ref, out_ref):
    parts = []
    for u in range(4):
        acc = msg_ref[u] + msg_ref[4 + u] + msg_ref[8 + u] + msg_ref[12 + u]
        parts.append(acc * 0.25)
    msgmean = jnp.concatenate(parts, axis=1)
    term2 = jnp.dot(b64_ref[...], wbm_ref[...],
                    preferred_element_type=jnp.float32)
    out_ref[...] = jnp.maximum(msgmean + term2 + bias_ref[...], 0.0)


def _finalize_layer(msg1r3, b64, wbmean, bias, bn=512):
    grid = (N_NODES // bn,)
    return pl.pallas_call(
        _k4_body,
        grid=grid,
        in_specs=[
            pl.BlockSpec((NQ, bn, CW), lambda i: (0, i, 0)),
            pl.BlockSpec((bn, 64), lambda i: (i, 0)),
            pl.BlockSpec((64, HIDDEN), lambda i: (0, 0)),
            pl.BlockSpec((1, HIDDEN), lambda i: (0, 0)),
        ],
        out_specs=pl.BlockSpec((bn, HIDDEN), lambda i: (i, 0)),
        out_shape=jax.ShapeDtypeStruct((N_NODES, HIDDEN), jnp.float32),
    )(msg1r3, b64, wbmean, bias)


def _gat_layer(x, srcb4, dstb4, eab, edge_attr, W, We, a_s, a_d, a_e, b):
    H, C = a_s.shape
    wf = W.reshape(W.shape[0], H * C)
    eye = jnp.eye(H, dtype=jnp.float32)
    asd_s = (a_s[:, None, :, None] * eye[:, :, None, None]).transpose(
        0, 2, 1, 3).reshape(H * C, H)
    asd_d = (a_d[:, None, :, None] * eye[:, :, None, None]).transpose(
        0, 2, 1, 3).reshape(H * C, H)
    z = jnp.zeros((H * C, 128 - H), jnp.float32)
    asd2 = jnp.concatenate([asd_s, z, asd_d, z], axis=1)

    h2, al2, cmax = _linear_logits(x, wf, asd2)
    als128 = al2[:, :128]
    ald128 = al2[:, 128:]

    ce = jnp.einsum('dhc,hc->dh', We, a_e)
    al_e = edge_attr @ ce
    ale_max = jnp.max(al_e, axis=0)

    craw = cmax[0, :H] + cmax[0, 128:128 + H] + ale_max
    cshift = jnp.where(craw > 0, craw, craw * 0.2)
    cvec = jnp.pad(cshift, (0, 12))

    aleb = jnp.pad(al_e, ((0, 0), (0, 12))).reshape(NBP4, 64, 16)

    alpha16 = _sc_softmax(als128, ald128, aleb, srcb4, dstb4, cvec)
    alpha = alpha16.reshape(N_EDGES, 16)[:, :H]

    arep = jnp.broadcast_to(
        alpha.T.reshape(H * NBP4, 64, 1), (H * NBP4, 64, 16))
    h2r = h2.reshape(NQ * N_NODES, CW)
    msg1r, bmatw = _sc_aggregate(h2r, srcb4, dstb4, arep, eab)

    b64 = bmatw.reshape(2, N_NODES, CW)[:, :, :32].transpose(
        1, 0, 2).reshape(N_NODES, 64)
    wep = jnp.pad(We, ((0, 16 - EDGE_DIM), (0, 0), (0, 0)))
    wbmean = wep.transpose(1, 0, 2).reshape(64, HIDDEN) * 0.25
    return _finalize_layer(msg1r.reshape(NQ, N_NODES, CW), b64, wbmean,
                           b.reshape(1, HIDDEN))


def kernel(x, edge_index, batch_index, edge_attr, params):
    src = edge_index[0]
    dst = edge_index[1]
    srcb4 = src.reshape(NBP4, 64)
    dstb4 = dst.reshape(NBP4, 64)
    eab = jnp.pad(edge_attr, ((0, 0), (0, 16 - EDGE_DIM))).reshape(
        NBP4, 64, 16)

    h = x
    for l in range(3):
        h = (_gat_layer(
            h, srcb4, dstb4, eab, edge_attr,
            params['W%d' % l], params['We%d' % l], params['as%d' % l],
            params['ad%d' % l], params['ae%d' % l], params['b%d' % l]))
    mu = jnp.mean(h, axis=0)
    var = jnp.var(h, axis=0)
    h = (h - mu) / jnp.sqrt(var + 1e-5) * params['bn_g'] + params['bn_b']
    gmax = jax.ops.segment_max(h, batch_index, num_segments=N_GRAPHS)
    gmax = jnp.where(jnp.isfinite(gmax), gmax, 0.0)
    counts = jax.ops.segment_sum(
        jnp.ones((h.shape[0], 1), jnp.float32), batch_index,
        num_segments=N_GRAPHS)
    gmean = jax.ops.segment_sum(
        h, batch_index, num_segments=N_GRAPHS) / jnp.maximum(counts, 1.0)
    g = jnp.concatenate([gmax, gmean], axis=1)
    g = jax.nn.relu(g @ params['fc1_W'] + params['fc1_b'])
    g = jax.nn.relu(g @ params['fc2_W'] + params['fc2_b'])
    return g @ params['fc3_W'] + params['fc3_b']


# speedup vs baseline: 4.6960x; 1.0877x over previous
import functools

import jax
import jax.numpy as jnp
import numpy as np
from jax import lax
from jax.experimental import pallas as pl
from jax.experimental.pallas import tpu as pltpu
from jax.experimental.pallas import tpu_sc as plsc

N_NODES = 10240
N_EDGES = 22528
NUM_FEATURES = 40
EDGE_DIM = 10
HIDDEN = 512
HEADS = 4
N_GRAPHS = 256
HC = HEADS * HIDDEN

NBP = N_EDGES // 16
NBP4 = N_EDGES // 64
NBP8 = N_EDGES // 128
CW = 128
NQ = HC // CW
NQC = NQ // 2
DR = N_NODES // 16


def _k1_body(x_ref, w_ref, asd_ref, h2_ref, al_ref, cmax_ref):
    i = pl.program_id(0)
    j = pl.program_id(1)
    h2 = jnp.dot(x_ref[...], w_ref[...], preferred_element_type=jnp.float32)
    bn = h2.shape[0]
    h2_ref[...] = jnp.transpose(h2.reshape(bn, 4, CW), (1, 0, 2))
    al = jnp.dot(h2, asd_ref[...], preferred_element_type=jnp.float32)

    @pl.when(j == 0)
    def _():
        al_ref[...] = jnp.zeros_like(al_ref)

    al_ref[...] += al

    @pl.when(j == pl.num_programs(1) - 1)
    def _():
        blkmax = jnp.max(al_ref[...], axis=0, keepdims=True)

        @pl.when(i == 0)
        def _():
            cmax_ref[...] = blkmax

        @pl.when(i > 0)
        def _():
            cmax_ref[...] = jnp.maximum(cmax_ref[...], blkmax)


def _linear_logits(x, wf, asd2, bn=512, bc=512):
    n, din = x.shape
    grid = (n // bn, HC // bc)
    return pl.pallas_call(
        _k1_body,
        grid=grid,
        in_specs=[
            pl.BlockSpec((bn, din), lambda i, j: (i, 0)),
            pl.BlockSpec((din, bc), lambda i, j: (0, j)),
            pl.BlockSpec((bc, 256), lambda i, j: (j, 0)),
        ],
        out_specs=[
            pl.BlockSpec((4, bn, CW), lambda i, j: (j, i, 0)),
            pl.BlockSpec((bn, 256), lambda i, j: (i, 0)),
            pl.BlockSpec((1, 256), lambda i, j: (0, 0)),
        ],
        out_shape=[
            jax.ShapeDtypeStruct((NQ, n, CW), jnp.float32),
            jax.ShapeDtypeStruct((n, 256), jnp.float32),
            jax.ShapeDtypeStruct((1, 256), jnp.float32),
        ],
    )(x, wf, asd2)


_GPT = NBP4 // 16
_GPT2 = NBP4 // 32


def _k2_body(als_hbm, ald_hbm, aleb_hbm, srcb_hbm, dstb_hbm, cvec_hbm,
             alpha_hbm, sidx_v, didx_v, gsrc_v, gdst_v, ale_v, exv_v, av_v,
             cvec_v, denom_sh, sem1, sem2):
    c = lax.axis_index("c")
    s = lax.axis_index("s")
    pltpu.sync_copy(cvec_hbm, cvec_v)
    iota16 = lax.broadcasted_iota(jnp.int32, (16,), 0)
    lanemask = jnp.where(iota16 < HEADS, 1.0, 0.0)
    cv = cvec_v[...]

    for i in range(64):
        def _zb(j, carry):
            exv_v[i, pl.ds(j * 16, 16)] = jnp.zeros((16,), jnp.float32)
            return carry
        lax.fori_loop(0, 8, _zb, 0)
    for k in range(DR // 64):
        pltpu.sync_copy(exv_v, denom_sh.at[pl.ds(s * DR + k * 64, 64)])
    plsc.subcore_barrier()

    def _ex_rows(g):
        cp1 = pltpu.async_copy(srcb_hbm.at[g], sidx_v, sem1)
        cp2 = pltpu.async_copy(dstb_hbm.at[g], didx_v, sem1)
        cp3 = pltpu.async_copy(aleb_hbm.at[g], ale_v, sem1)
        cp1.wait()
        cp2.wait()
        cp3.wait()
        cp4 = pltpu.async_copy(als_hbm.at[sidx_v], gsrc_v, sem2)
        cp5 = pltpu.async_copy(ald_hbm.at[didx_v], gdst_v, sem2)
        cp4.wait()
        cp5.wait()
        for e in range(64):
            srow = (gsrc_v[e, pl.ds(0, 16)] + gdst_v[e, pl.ds(0, 16)]
                    + ale_v[e, :])
            srow = jnp.where(srow > 0, srow, srow * 0.2)
            exv_v[e, pl.ds(0, 16)] = jnp.exp(srow - cv) * lanemask

    def _p1(i, carry):
        g = s + i * 16
        _ex_rows(g)
        pltpu.sync_copy(exv_v, denom_sh.at[didx_v], add=True)
        return carry

    lax.fori_loop(0, _GPT, _p1, 0)
    plsc.subcore_barrier()

    def _p2(i, carry):
        g = c * (NBP4 // 2) + s + i * 16
        _ex_rows(g)
        pltpu.sync_copy(denom_sh.at[didx_v], gsrc_v)
        for e in range(64):
            d = jnp.maximum(gsrc_v[e, pl.ds(0, 16)], 1e-16)
            av_v[e, :] = exv_v[e, pl.ds(0, 16)] / d
        pltpu.sync_copy(av_v, alpha_hbm.at[g])
        return carry

    lax.fori_loop(0, _GPT2, _p2, 0)


def _sc_softmax(als128, ald128, aleb, srcb4, dstb4, cvec):
    mesh = plsc.VectorSubcoreMesh(core_axis_name="c", subcore_axis_name="s")
    return pl.kernel(
        _k2_body,
        out_type=jax.ShapeDtypeStruct((NBP4, 64, 16), jnp.float32),
        mesh=mesh,
        scratch_types=[
            pltpu.VMEM((64,), jnp.int32),
            pltpu.VMEM((64,), jnp.int32),
            pltpu.VMEM((64, 128), jnp.float32),
            pltpu.VMEM((64, 128), jnp.float32),
            pltpu.VMEM((64, 16), jnp.float32),
            pltpu.VMEM((64, 128), jnp.float32),
            pltpu.VMEM((64, 16), jnp.float32),
            pltpu.VMEM((16,), jnp.float32),
            pltpu.VMEM_SHARED((N_NODES, 128), jnp.float32),
            pltpu.SemaphoreType.DMA,
            pltpu.SemaphoreType.DMA,
        ],
    )(als128, ald128, aleb, srcb4, dstb4, cvec)


_BPT8 = NBP8 // 16


def _k3_body(h2r_hbm, srcb_hbm, dstb_hbm, arep_hbm, eab_hbm, zros_hbm,
             out_hbm, bmatw_hbm,
             rows_v, sidx_v, didx_v, arr_v,
             acc_sh, sem1, sem2):
    c = lax.axis_index("c")
    s = lax.axis_index("s")

    def _zero_acc():
        pltpu.sync_copy(zros_hbm, acc_sh.at[pl.ds(s * DR, DR)])

    _zero_acc()
    plsc.subcore_barrier()

    def _qpass(q, qcarry):
        qg = c * NQC + q
        hq = qg // (NQ // HEADS)

        def _batch(i, carry):
            g = s + i * 16
            cp1 = pltpu.async_copy(srcb_hbm.at[g], sidx_v, sem1)
            cp2 = pltpu.async_copy(dstb_hbm.at[g], didx_v, sem1)
            cp3 = pltpu.async_copy(arep_hbm.at[hq * NBP8 + g], arr_v, sem1)
            cp1.wait()
            cp2.wait()
            cp3.wait()
            for j in range(8):
                sl = pl.ds(j * 16, 16)
                sidx_v[sl] = sidx_v[sl] + qg * N_NODES
            pltpu.async_copy(h2r_hbm.at[sidx_v], rows_v, sem2).wait()
            for e in range(128):
                av = arr_v[e, :]
                for u in range(CW // 16):
                    rows_v[e, pl.ds(u * 16, 16)] = (
                        rows_v[e, pl.ds(u * 16, 16)] * av)
            pltpu.sync_copy(rows_v, acc_sh.at[didx_v], add=True)
            return carry

        lax.fori_loop(0, _BPT8, _batch, 0)
        plsc.subcore_barrier()
        pltpu.sync_copy(acc_sh.at[pl.ds(s * DR, DR)],
                        out_hbm.at[pl.ds(qg * N_NODES + s * DR, DR)])
        _zero_acc()
        plsc.subcore_barrier()
        return qcarry

    lax.fori_loop(0, NQC, _qpass, 0)

    h0 = 2 * c

    def _bbatch(i, carry):
        g = s + i * 16
        cp1 = pltpu.async_copy(dstb_hbm.at[g], didx_v, sem1)
        cp4 = pltpu.async_copy(eab_hbm.at[g], arr_v, sem1)
        cp1.wait()
        cp4.wait()
        for e in range(128):
            rows_v[e, pl.ds(32, 16)] = arr_v[e, :]
        cp2 = pltpu.async_copy(arep_hbm.at[h0 * NBP8 + g], arr_v, sem1)
        cp2.wait()
        for e in range(128):
            rows_v[e, pl.ds(0, 16)] = rows_v[e, pl.ds(32, 16)] * arr_v[e, :]
        cp3 = pltpu.async_copy(arep_hbm.at[(h0 + 1) * NBP8 + g], arr_v, sem1)
        cp3.wait()
        for e in range(128):
            rows_v[e, pl.ds(16, 16)] = rows_v[e, pl.ds(32, 16)] * arr_v[e, :]
        pltpu.sync_copy(rows_v, acc_sh.at[didx_v], add=True)
        return carry

    lax.fori_loop(0, _BPT8, _bbatch, 0)
    plsc.subcore_barrier()
    pltpu.sync_copy(acc_sh.at[pl.ds(s * DR, DR)],
                    bmatw_hbm.at[pl.ds(c * N_NODES + s * DR, DR)])


def _sc_aggregate(h2r, srcb8, dstb8, arep, eab):
    mesh = plsc.VectorSubcoreMesh(core_axis_name="c", subcore_axis_name="s")
    return pl.kernel(
        _k3_body,
        out_type=(
            jax.ShapeDtypeStruct((NQ * N_NODES, CW), jnp.float32),
            jax.ShapeDtypeStruct((2 * N_NODES, CW), jnp.float32),
        ),
        mesh=mesh,
        scratch_types=[
            pltpu.VMEM((128, CW), jnp.float32),
            pltpu.VMEM((128,), jnp.int32),
            pltpu.VMEM((128,), jnp.int32),
            pltpu.VMEM((128, 16), jnp.float32),
            pltpu.VMEM_SHARED((N_NODES, CW), jnp.float32),
            pltpu.SemaphoreType.DMA,
            pltpu.SemaphoreType.DMA,
        ],
    )(h2r, srcb8, dstb8, arep, eab, jnp.zeros((DR, CW), jnp.float32))


def _k4_body(msg_ref, b64_ref, wbm_ref, bias_ref, out_ref):
    parts = []
    for u in range(4):
        acc = msg_ref[u] + msg_ref[4 + u] + msg_ref[8 + u] + msg_ref[12 + u]
        parts.append(acc * 0.25)
    msgmean = jnp.concatenate(parts, axis=1)
    term2 = jnp.dot(b64_ref[...], wbm_ref[...],
                    preferred_element_type=jnp.float32)
    out_ref[...] = jnp.maximum(msgmean + term2 + bias_ref[...], 0.0)


def _finalize_layer(msg1r3, b64, wbmean, bias, bn=512):
    grid = (N_NODES // bn,)
    return pl.pallas_call(
        _k4_body,
        grid=grid,
        in_specs=[
            pl.BlockSpec((NQ, bn, CW), lambda i: (0, i, 0)),
            pl.BlockSpec((bn, 64), lambda i: (i, 0)),
            pl.BlockSpec((64, HIDDEN), lambda i: (0, 0)),
            pl.BlockSpec((1, HIDDEN), lambda i: (0, 0)),
        ],
        out_specs=pl.BlockSpec((bn, HIDDEN), lambda i: (i, 0)),
        out_shape=jax.ShapeDtypeStruct((N_NODES, HIDDEN), jnp.float32),
    )(msg1r3, b64, wbmean, bias)


def _gat_layer(x, srcb4, dstb4, srcb8, dstb8, eab8, edge_attr,
               W, We, a_s, a_d, a_e, b):
    H, C = a_s.shape
    wf = W.reshape(W.shape[0], H * C)
    eye = jnp.eye(H, dtype=jnp.float32)
    asd_s = (a_s[:, None, :, None] * eye[:, :, None, None]).transpose(
        0, 2, 1, 3).reshape(H * C, H)
    asd_d = (a_d[:, None, :, None] * eye[:, :, None, None]).transpose(
        0, 2, 1, 3).reshape(H * C, H)
    z = jnp.zeros((H * C, 128 - H), jnp.float32)
    asd2 = jnp.concatenate([asd_s, z, asd_d, z], axis=1)

    h2, al2, cmax = _linear_logits(x, wf, asd2)
    als128 = al2[:, :128]
    ald128 = al2[:, 128:]

    ce = jnp.einsum('dhc,hc->dh', We, a_e)
    al_e = edge_attr @ ce
    ale_max = jnp.max(al_e, axis=0)

    craw = cmax[0, :H] + cmax[0, 128:128 + H] + ale_max
    cshift = jnp.where(craw > 0, craw, craw * 0.2)
    cvec = jnp.pad(cshift, (0, 12))

    aleb = jnp.pad(al_e, ((0, 0), (0, 12))).reshape(NBP4, 64, 16)

    alpha16 = _sc_softmax(als128, ald128, aleb, srcb4, dstb4, cvec)
    alpha = alpha16.reshape(N_EDGES, 16)[:, :H]

    arep = jnp.broadcast_to(
        alpha.T.reshape(H * NBP8, 128, 1), (H * NBP8, 128, 16))
    h2r = h2.reshape(NQ * N_NODES, CW)
    msg1r, bmatw = _sc_aggregate(h2r, srcb8, dstb8, arep, eab8)

    b64 = bmatw.reshape(2, N_NODES, CW)[:, :, :32].transpose(
        1, 0, 2).reshape(N_NODES, 64)
    wep = jnp.pad(We, ((0, 16 - EDGE_DIM), (0, 0), (0, 0)))
    wbmean = wep.transpose(1, 0, 2).reshape(64, HIDDEN) * 0.25
    return _finalize_layer(msg1r.reshape(NQ, N_NODES, CW), b64, wbmean,
                           b.reshape(1, HIDDEN))


def kernel(x, edge_index, batch_index, edge_attr, params):
    src = edge_index[0]
    dst = edge_index[1]
    srcb4 = src.reshape(NBP4, 64)
    dstb4 = dst.reshape(NBP4, 64)
    srcb8 = src.reshape(NBP8, 128)
    dstb8 = dst.reshape(NBP8, 128)
    eab8 = jnp.pad(edge_attr, ((0, 0), (0, 16 - EDGE_DIM))).reshape(
        NBP8, 128, 16)

    h = x
    for l in range(3):
        h = (_gat_layer(
            h, srcb4, dstb4, srcb8, dstb8, eab8, edge_attr,
            params['W%d' % l], params['We%d' % l], params['as%d' % l],
            params['ad%d' % l], params['ae%d' % l], params['b%d' % l]))
    mu = jnp.mean(h, axis=0)
    var = jnp.var(h, axis=0)
    h = (h - mu) / jnp.sqrt(var + 1e-5) * params['bn_g'] + params['bn_b']
    gmax = jax.ops.segment_max(h, batch_index, num_segments=N_GRAPHS)
    gmax = jnp.where(jnp.isfinite(gmax), gmax, 0.0)
    counts = jax.ops.segment_sum(
        jnp.ones((h.shape[0], 1), jnp.float32), batch_index,
        num_segments=N_GRAPHS)
    gmean = jax.ops.segment_sum(
        h, batch_index, num_segments=N_GRAPHS) / jnp.maximum(counts, 1.0)
    g = jnp.concatenate([gmax, gmean], axis=1)
    g = jax.nn.relu(g @ params['fc1_W'] + params['fc1_b'])
    g = jax.nn.relu(g @ params['fc2_W'] + params['fc2_b'])
    return g @ params['fc3_W'] + params['fc3_b']
